# Initial kernel scaffold; baseline (speedup 1.0000x reference)
import functools

import jax
import jax.numpy as jnp
from jax import lax
from jax.experimental import pallas as pl
from jax.experimental.pallas import tpu as pltpu
from jax.experimental.pallas import tpu_sc as plsc

VOCAB = 1000012
B = 16384
NF = 26
D = 16

NC = 2
NS = 16
NW = NC * NS
BPW = B // NW
IPW = BPW * NF
GROWS = 128
CH = 64
GPC = CH * NF // GROWS
NCHUNK = BPW // CH


def _sc_body(ids_hbm, rat_hbm, table_hbm, bias_hbm, out_hbm,
             idx_v, rat_v, rows_v, out_v, bias_v, sem):
    wid = lax.axis_index("s") * NC + lax.axis_index("c")

    pltpu.sync_copy(ids_hbm.at[wid], idx_v)
    pltpu.sync_copy(rat_hbm.at[wid], rat_v)
    pltpu.sync_copy(bias_hbm, bias_v)
    bvec = bias_v[...]
    zvec = jnp.zeros((16,), jnp.float32)

    def chunk_body(c, carry):
        copies = []
        for g in range(GPC):
            copies.append(pltpu.async_copy(
                table_hbm.at[idx_v.at[c * GPC + g]],
                rows_v.at[pl.ds(g * GROWS, GROWS)],
                sem))
        for cp in copies:
            cp.wait()

        def sample_body(s, carry2):
            gbase = (c * CH + s) * NF
            accs = [bvec, zvec, zvec, zvec]
            for f in range(NF):
                rb = plsc.load_gather(
                    rat_v, [jnp.broadcast_to(jnp.int32(gbase + f), (16,))])
                row = rows_v[s * NF + f]
                accs[f % 4] = accs[f % 4] + row * rb
            out_v[c * CH + s] = (accs[0] + accs[1]) + (accs[2] + accs[3])
            return carry2

        return lax.fori_loop(0, CH, sample_body, carry)

    lax.fori_loop(0, NCHUNK, chunk_body, 0)
    pltpu.sync_copy(out_v, out_hbm.at[wid])


def kernel(feature_ids, feature_ratings, fc_weight, bias):
    ids3 = feature_ids.reshape(NW, IPW // GROWS, GROWS)
    rat2 = feature_ratings.reshape(NW, IPW)

    sc_call = functools.partial(
        pl.kernel,
        out_type=jax.ShapeDtypeStruct((NW, BPW, D), jnp.float32),
        mesh=plsc.VectorSubcoreMesh(core_axis_name="c", subcore_axis_name="s"),
        scratch_types=[
            pltpu.VMEM((IPW // GROWS, GROWS), jnp.int32),
            pltpu.VMEM((IPW,), jnp.float32),
            pltpu.VMEM((CH * NF, D), jnp.float32),
            pltpu.VMEM((BPW, D), jnp.float32),
            pltpu.VMEM((D,), jnp.float32),
            pltpu.SemaphoreType.DMA,
        ],
    )(_sc_body)

    out = sc_call(ids3, rat2, fc_weight, bias)
    return out.reshape(B, D)


# speedup vs baseline: 1.0176x; 1.0176x over previous
import functools

import jax
import jax.numpy as jnp
from jax import lax
from jax.experimental import pallas as pl
from jax.experimental.pallas import tpu as pltpu
from jax.experimental.pallas import tpu_sc as plsc

VOCAB = 1000012
B = 16384
NF = 26
D = 16

NC = 2
NS = 16
NW = NC * NS
BPW = B // NW
IPW = BPW * NF
GROWS = 128
CH = 64
GPC = CH * NF // GROWS
NCHUNK = BPW // CH


def _lane_broadcast(vec, lane):
    idx = jnp.full((16, 1), lane, jnp.int32)
    dnums = lax.GatherDimensionNumbers(
        offset_dims=(), collapsed_slice_dims=(0,), start_index_map=(0,))
    return lax.gather(vec, idx, dnums, (1,),
                      mode=lax.GatherScatterMode.PROMISE_IN_BOUNDS)


def _sc_body(ids_hbm, rat_hbm, table_hbm, bias_hbm, out_hbm,
             idx_v, rat_v, rows_v, out_v, bias_v, sem):
    wid = lax.axis_index("s") * NC + lax.axis_index("c")

    pltpu.sync_copy(ids_hbm.at[wid], idx_v)
    pltpu.sync_copy(rat_hbm.at[wid], rat_v)
    pltpu.sync_copy(bias_hbm, bias_v)
    bvec = bias_v[...]
    zvec = jnp.zeros((16,), jnp.float32)

    def chunk_body(c, carry):
        for g in range(GPC):
            pltpu.async_copy(
                table_hbm.at[idx_v.at[c * GPC + g]],
                rows_v.at[pl.ds(g * GROWS, GROWS)],
                sem).wait()

        def sample_body(s, carry2):
            gbase = (c * CH + s) * 32
            rv0 = rat_v[pl.ds(gbase, 16)]
            rv1 = rat_v[pl.ds(gbase + 16, 16)]
            accs = [bvec, zvec, zvec, zvec]
            for f in range(NF):
                rv = rv0 if f < 16 else rv1
                rb = _lane_broadcast(rv, f % 16)
                row = rows_v[s * NF + f]
                accs[f % 4] = accs[f % 4] + row * rb
            out_v[c * CH + s] = (accs[0] + accs[1]) + (accs[2] + accs[3])
            return carry2

        return lax.fori_loop(0, CH, sample_body, carry)

    lax.fori_loop(0, NCHUNK, chunk_body, 0)
    pltpu.sync_copy(out_v, out_hbm.at[wid])


def kernel(feature_ids, feature_ratings, fc_weight, bias):
    ids3 = feature_ids.reshape(NW, IPW // GROWS, GROWS)
    rat_pad = jnp.pad(feature_ratings, ((0, 0), (0, 32 - NF)))
    rat2 = rat_pad.reshape(NW, BPW * 32)

    sc_call = functools.partial(
        pl.kernel,
        out_type=jax.ShapeDtypeStruct((NW, BPW, D), jnp.float32),
        mesh=plsc.VectorSubcoreMesh(core_axis_name="c", subcore_axis_name="s"),
        compiler_params=pltpu.CompilerParams(use_tc_tiling_on_sc=False),
        scratch_types=[
            pltpu.VMEM((IPW // GROWS, GROWS), jnp.int32),
            pltpu.VMEM((BPW * 32,), jnp.float32),
            pltpu.VMEM((CH * NF, D), jnp.float32),
            pltpu.VMEM((BPW, D), jnp.float32),
            pltpu.VMEM((D,), jnp.float32),
            pltpu.SemaphoreType.DMA,
        ],
    )(_sc_body)

    out = sc_call(ids3, rat2, fc_weight, bias)
    return out.reshape(B, D)


# speedup vs baseline: 1.2951x; 1.2727x over previous
import functools

import jax
import jax.numpy as jnp
from jax import lax
from jax.experimental import pallas as pl
from jax.experimental.pallas import tpu as pltpu
from jax.experimental.pallas import tpu_sc as plsc

VOCAB = 1000012
B = 16384
NF = 26
D = 16

NC = 2
NS = 16
NW = NC * NS
BPW = B // NW
IPW = BPW * NF

NFULL = VOCAB // 128
NTAIL = VOCAB - NFULL * 128
NSLOT = NFULL // NW + 1

CH = 16
RPC = CH * NF
GROWS = 104
NCHUNK = BPW // CH


def _lane_broadcast(vec, lane):
    idx = jnp.full((16, 1), lane, jnp.int32)
    dnums = lax.GatherDimensionNumbers(
        offset_dims=(), collapsed_slice_dims=(0,), start_index_map=(0,))
    return lax.gather(vec, idx, dnums, (1,),
                      mode=lax.GatherScatterMode.PROMISE_IN_BOUNDS)


def _transpose_block(in_ref, out_ref, nrows):
    iota = lax.broadcasted_iota(jnp.int32, (16,), 0)
    for j in range(nrows):
        row = plsc.load_gather(
            in_ref, [iota, jnp.full((16,), j, jnp.int32)])
        out_ref[pl.ds(16 * j, 16)] = row


def _stage1_body(tview_hbm, tail_hbm, out_hbm, in_v0, in_v1, out_v0, out_v1,
                 sem_in, sem_out):
    wid = lax.axis_index("s") * NC + lax.axis_index("c")
    in_bufs = (in_v0, in_v1)
    out_bufs = (out_v0, out_v1)

    def issue_in(t, b):
        cid = t * NW + wid
        pltpu.async_copy(tview_hbm.at[:, pl.ds(cid * 128, 128)],
                         in_bufs[b], sem_in.at[b])

    def wait_in(b):
        pltpu.make_async_copy(tview_hbm.at[:, pl.ds(0, 128)],
                              in_bufs[b], sem_in.at[b]).wait()

    def issue_out(cid, b):
        pltpu.async_copy(out_bufs[b],
                         out_hbm.at[pl.ds(cid * 2048, 2048)], sem_out.at[b])

    def wait_out(b):
        pltpu.make_async_copy(out_bufs[b],
                              out_hbm.at[pl.ds(0, 2048)], sem_out.at[b]).wait()

    issue_in(0, 0)
    issue_in(1, 1)

    def loop_body(tt, carry):
        for b in range(2):
            t = 2 * tt + b
            cid = t * NW + wid

            @pl.when(cid < NFULL)
            def _():
                wait_in(b)

                @pl.when(t >= 2)
                def _():
                    wait_out(b)

                _transpose_block(in_bufs[b], out_bufs[b], 128)
                issue_out(cid, b)

            @pl.when((t + 2) * NW + wid < NFULL)
            def _():
                issue_in(t + 2, b)
        return carry

    lax.fori_loop(0, (NSLOT + 1) // 2, loop_body, 0)
    wait_out(0)
    wait_out(1)

    @pl.when(wid == NW - 1)
    def _():
        pltpu.sync_copy(tail_hbm, in_v0)
        _transpose_block(in_v0, out_v0, NTAIL)
        pltpu.sync_copy(out_v0.at[pl.ds(0, NTAIL * 16)],
                        out_hbm.at[pl.ds(NFULL * 2048, NTAIL * 16)])


def _stage2_body(ids_hbm, rat_hbm, table_hbm, bias_hbm, out_hbm,
                 idx_v, rat_v, rows_v, out_v, bias_v, sems):
    wid = lax.axis_index("s") * NC + lax.axis_index("c")

    pltpu.sync_copy(ids_hbm.at[wid], idx_v)
    pltpu.sync_copy(rat_hbm.at[wid], rat_v)
    pltpu.sync_copy(bias_hbm, bias_v)
    bvec = bias_v[...]
    zvec = jnp.zeros((16,), jnp.float32)

    def issue(chunk, buf):
        for g in range(4):
            pltpu.async_copy(
                table_hbm.at[idx_v.at[4 * chunk + g]],
                rows_v.at[pl.ds(buf * RPC + g * GROWS, GROWS)],
                sems.at[buf])

    def drain(buf):
        for g in range(4):
            pltpu.make_async_copy(
                table_hbm.at[idx_v.at[g]],
                rows_v.at[pl.ds(buf * RPC + g * GROWS, GROWS)],
                sems.at[buf]).wait()

    issue(0, 0)
    issue(1, 1)

    def compute(chunk, buf):
        base = buf * RPC

        def sample_body(s, carry2):
            gbase = (chunk * CH + s) * 32
            rv0 = rat_v[pl.ds(gbase, 16)]
            rv1 = rat_v[pl.ds(gbase + 16, 16)]
            accs = [bvec, zvec, zvec, zvec]
            for f in range(NF):
                rv = rv0 if f < 16 else rv1
                rb = _lane_broadcast(rv, f % 16)
                row = rows_v[base + s * NF + f]
                accs[f % 4] = accs[f % 4] + row * rb
            out_v[chunk * CH + s] = (accs[0] + accs[1]) + (accs[2] + accs[3])
            return carry2

        lax.fori_loop(0, CH, sample_body, 0)

    def loop_body(tt, carry):
        for b in range(2):
            c = 2 * tt + b
            drain(b)
            compute(c, b)

            @pl.when(c + 2 < NCHUNK)
            def _():
                issue(c + 2, b)
        return carry

    lax.fori_loop(0, NCHUNK // 2, loop_body, 0)
    pltpu.sync_copy(out_v, out_hbm.at[wid])


def kernel(feature_ids, feature_ratings, fc_weight, bias):
    tview = fc_weight.T
    tail128 = jnp.pad(fc_weight[NFULL * 128:].T, ((0, 0), (0, 128 - NTAIL)))

    stage1 = functools.partial(
        pl.kernel,
        out_type=jax.ShapeDtypeStruct((VOCAB * D,), jnp.float32),
        mesh=plsc.VectorSubcoreMesh(core_axis_name="c", subcore_axis_name="s"),
        compiler_params=pltpu.CompilerParams(
            use_tc_tiling_on_sc=True, needs_layout_passes=False),
        scratch_types=[
            pltpu.VMEM((16, 128), jnp.float32),
            pltpu.VMEM((16, 128), jnp.float32),
            pltpu.VMEM((2048,), jnp.float32),
            pltpu.VMEM((2048,), jnp.float32),
            pltpu.SemaphoreType.DMA((2,)),
            pltpu.SemaphoreType.DMA((2,)),
        ],
    )(_stage1_body)

    table_rm = stage1(tview, tail128).reshape(VOCAB, D)

    ids3 = feature_ids.reshape(NW, 4 * NCHUNK, GROWS)
    rat2 = jnp.pad(feature_ratings, ((0, 0), (0, 32 - NF))).reshape(NW, BPW * 32)

    stage2 = functools.partial(
        pl.kernel,
        out_type=jax.ShapeDtypeStruct((NW, BPW, D), jnp.float32),
        mesh=plsc.VectorSubcoreMesh(core_axis_name="c", subcore_axis_name="s"),
        compiler_params=pltpu.CompilerParams(use_tc_tiling_on_sc=False),
        scratch_types=[
            pltpu.VMEM((4 * NCHUNK, GROWS), jnp.int32),
            pltpu.VMEM((BPW * 32,), jnp.float32),
            pltpu.VMEM((2 * RPC, D), jnp.float32),
            pltpu.VMEM((BPW, D), jnp.float32),
            pltpu.VMEM((D,), jnp.float32),
            pltpu.SemaphoreType.DMA((2,)),
        ],
    )(_stage2_body)

    out = stage2(ids3, rat2, table_rm, bias)
    return out.reshape(B, D)


# speedup vs baseline: 2.4967x; 1.9278x over previous
import functools

import jax
import jax.numpy as jnp
from jax import lax
from jax.experimental import pallas as pl
from jax.experimental.pallas import tpu as pltpu
from jax.experimental.pallas import tpu_sc as plsc

VOCAB = 1000012
B = 16384
NF = 26
D = 16

NC = 2
NS = 16
NW = NC * NS
BPW = B // NW
IPW = BPW * NF

NFULL = VOCAB // 128
NTAIL = VOCAB - NFULL * 128
NSLOT = NFULL // NW + 1

CH = 16
RPC = CH * NF
GROWS = 104
NCHUNK = BPW // CH


def _lane_broadcast(vec, lane):
    idx = jnp.full((16, 1), lane, jnp.int32)
    dnums = lax.GatherDimensionNumbers(
        offset_dims=(), collapsed_slice_dims=(0,), start_index_map=(0,))
    return lax.gather(vec, idx, dnums, (1,),
                      mode=lax.GatherScatterMode.PROMISE_IN_BOUNDS)


def _transpose_block(in_ref, out_ref):
    iota = lax.broadcasted_iota(jnp.int32, (16,), 0)
    for j in range(128):
        col = (iota + j) & 127
        vals = plsc.load_gather(in_ref, [iota, col])
        plsc.store_scatter(out_ref, [col * 16 + iota], vals)


def _stage1_body(tview_hbm, tail_hbm, out_hbm, in_v0, in_v1, out_v0, out_v1,
                 sem_in, sem_out):
    wid = lax.axis_index("s") * NC + lax.axis_index("c")
    in_bufs = (in_v0, in_v1)
    out_bufs = (out_v0, out_v1)

    def issue_in(t, b):
        cid = t * NW + wid
        pltpu.async_copy(tview_hbm.at[:, pl.ds(cid * 128, 128)],
                         in_bufs[b], sem_in.at[b])

    def wait_in(b):
        pltpu.make_async_copy(tview_hbm.at[:, pl.ds(0, 128)],
                              in_bufs[b], sem_in.at[b]).wait()

    def issue_out(cid, b):
        pltpu.async_copy(out_bufs[b],
                         out_hbm.at[pl.ds(cid * 2048, 2048)], sem_out.at[b])

    def wait_out(b):
        pltpu.make_async_copy(out_bufs[b],
                              out_hbm.at[pl.ds(0, 2048)], sem_out.at[b]).wait()

    issue_in(0, 0)
    issue_in(1, 1)

    def loop_body(tt, carry):
        for b in range(2):
            t = 2 * tt + b
            cid = t * NW + wid

            @pl.when(cid < NFULL)
            def _():
                wait_in(b)

                @pl.when(t >= 2)
                def _():
                    wait_out(b)

                _transpose_block(in_bufs[b], out_bufs[b])
                issue_out(cid, b)

            @pl.when((t + 2) * NW + wid < NFULL)
            def _():
                issue_in(t + 2, b)
        return carry

    lax.fori_loop(0, (NSLOT + 1) // 2, loop_body, 0)
    wait_out(0)
    wait_out(1)

    @pl.when(wid == NW - 1)
    def _():
        pltpu.sync_copy(tail_hbm, in_v0)
        _transpose_block(in_v0, out_v0)
        pltpu.sync_copy(out_v0.at[pl.ds(0, NTAIL * 16)],
                        out_hbm.at[pl.ds(NFULL * 2048, NTAIL * 16)])


def _stage2_body(ids_hbm, rat_hbm, table_hbm, bias_hbm, out_hbm,
                 idx_v, rat_v, rows_v, out_v, bias_v, sems):
    wid = lax.axis_index("s") * NC + lax.axis_index("c")

    pltpu.sync_copy(ids_hbm.at[wid], idx_v)
    pltpu.sync_copy(rat_hbm.at[wid], rat_v)
    pltpu.sync_copy(bias_hbm, bias_v)
    bvec = bias_v[...]
    zvec = jnp.zeros((16,), jnp.float32)

    def issue(chunk, buf):
        for g in range(4):
            pltpu.async_copy(
                table_hbm.at[idx_v.at[4 * chunk + g]],
                rows_v.at[pl.ds(buf * RPC + g * GROWS, GROWS)],
                sems.at[buf])

    def drain(buf):
        for g in range(4):
            pltpu.make_async_copy(
                table_hbm.at[idx_v.at[g]],
                rows_v.at[pl.ds(buf * RPC + g * GROWS, GROWS)],
                sems.at[buf]).wait()

    issue(0, 0)
    issue(1, 1)

    def compute(chunk, buf):
        base = buf * RPC

        def sample_body(s, carry2):
            gbase = (chunk * CH + s) * 32
            rv0 = rat_v[pl.ds(gbase, 16)]
            rv1 = rat_v[pl.ds(gbase + 16, 16)]
            accs = [bvec, zvec, zvec, zvec]
            for f in range(NF):
                rv = rv0 if f < 16 else rv1
                rb = _lane_broadcast(rv, f % 16)
                row = rows_v[base + s * NF + f]
                accs[f % 4] = accs[f % 4] + row * rb
            out_v[chunk * CH + s] = (accs[0] + accs[1]) + (accs[2] + accs[3])
            return carry2

        lax.fori_loop(0, CH, sample_body, 0)

    def loop_body(tt, carry):
        for b in range(2):
            c = 2 * tt + b
            drain(b)
            compute(c, b)

            @pl.when(c + 2 < NCHUNK)
            def _():
                issue(c + 2, b)
        return carry

    lax.fori_loop(0, NCHUNK // 2, loop_body, 0)
    pltpu.sync_copy(out_v, out_hbm.at[wid])


def kernel(feature_ids, feature_ratings, fc_weight, bias):
    tview = fc_weight.T
    tail128 = jnp.pad(fc_weight[NFULL * 128:].T, ((0, 0), (0, 128 - NTAIL)))

    stage1 = functools.partial(
        pl.kernel,
        out_type=jax.ShapeDtypeStruct((VOCAB * D,), jnp.float32),
        mesh=plsc.VectorSubcoreMesh(core_axis_name="c", subcore_axis_name="s"),
        compiler_params=pltpu.CompilerParams(
            use_tc_tiling_on_sc=True, needs_layout_passes=False),
        scratch_types=[
            pltpu.VMEM((16, 128), jnp.float32),
            pltpu.VMEM((16, 128), jnp.float32),
            pltpu.VMEM((2048,), jnp.float32),
            pltpu.VMEM((2048,), jnp.float32),
            pltpu.SemaphoreType.DMA((2,)),
            pltpu.SemaphoreType.DMA((2,)),
        ],
    )(_stage1_body)

    table_rm = stage1(tview, tail128).reshape(VOCAB, D)

    ids3 = feature_ids.reshape(NW, 4 * NCHUNK, GROWS)
    rat2 = jnp.pad(feature_ratings, ((0, 0), (0, 32 - NF))).reshape(NW, BPW * 32)

    stage2 = functools.partial(
        pl.kernel,
        out_type=jax.ShapeDtypeStruct((NW, BPW, D), jnp.float32),
        mesh=plsc.VectorSubcoreMesh(core_axis_name="c", subcore_axis_name="s"),
        compiler_params=pltpu.CompilerParams(use_tc_tiling_on_sc=False),
        scratch_types=[
            pltpu.VMEM((4 * NCHUNK, GROWS), jnp.int32),
            pltpu.VMEM((BPW * 32,), jnp.float32),
            pltpu.VMEM((2 * RPC, D), jnp.float32),
            pltpu.VMEM((BPW, D), jnp.float32),
            pltpu.VMEM((D,), jnp.float32),
            pltpu.SemaphoreType.DMA((2,)),
        ],
    )(_stage2_body)

    out = stage2(ids3, rat2, table_rm, bias)
    return out.reshape(B, D)
